# baseline (speedup 1.0000x reference)
import jax
import jax.numpy as jnp
from jax import lax
from jax.experimental import pallas as pl
from jax.experimental.pallas import tpu as pltpu
from jax.experimental.pallas import tpu_sc as plsc

_B = 4096
_CTX = 20
_S = 21
_EMB = 64
_NC = 2
_NS = 16
_NW = _NC * _NS
_BPW = _B // _NW


def _body(d_hbm, w_hbm, wpt_hbm, docs_hbm, ctxs_hbm, y_hbm, out_hbm,
          idx_v, ident_v, rows_v, h_v, outc_v, acc_spm, sem):
  sid = lax.axis_index("s")
  wid = sid * _NC + lax.axis_index("c")
  base = wid * _BPW
  iota = lax.iota(jnp.int32, 16)
  sbase = sid * _BPW

  for j in range(_BPW // 16):
    ident_v[pl.ds(j * 16, 16)] = iota + (sbase + j * 16)

  pltpu.sync_copy(docs_hbm.at[pl.ds(base, _BPW)], idx_v)
  pltpu.async_copy(d_hbm.at[idx_v], rows_v, sem).wait()
  pltpu.sync_copy(rows_v, acc_spm.at[pl.ds(sbase, _BPW)])

  for c in range(_CTX):
    pltpu.sync_copy(ctxs_hbm.at[pl.ds(c * _B + base, _BPW)], idx_v)
    pltpu.async_copy(w_hbm.at[idx_v], rows_v, sem).wait()
    pltpu.sync_copy(rows_v, acc_spm.at[ident_v], add=True)

  pltpu.sync_copy(acc_spm.at[pl.ds(sbase, _BPW)], h_v)

  lane15 = iota == 15
  for s in range(_S):
    pltpu.sync_copy(y_hbm.at[pl.ds(s * _B + base, _BPW)], idx_v)
    pltpu.async_copy(wpt_hbm.at[idx_v], rows_v, sem).wait()

    def dot_row(i, _):
      acc = h_v[i, pl.ds(0, 16)] * rows_v[i, pl.ds(0, 16)]
      for j in range(1, _EMB // 16):
        sl = pl.ds(j * 16, 16)
        acc = acc + h_v[i, sl] * rows_v[i, sl]
      csum = plsc.cumsum(acc)
      plsc.store_scatter(outc_v, [jnp.zeros((16,), jnp.int32) + i], csum,
                         mask=lane15)
      return 0
    lax.fori_loop(0, _BPW, dot_row, 0)

    pltpu.sync_copy(outc_v, out_hbm.at[pl.ds(s * _B + base, _BPW)])


@jax.jit
def _dmm_call(d, w, wpt, docs, ctxs_t, y_t):
  mesh = plsc.VectorSubcoreMesh(
      core_axis_name="c", subcore_axis_name="s",
      num_cores=_NC, num_subcores=_NS)
  return pl.kernel(
      _body,
      out_type=jax.ShapeDtypeStruct((_S * _B,), jnp.float32),
      mesh=mesh,
      compiler_params=pltpu.CompilerParams(needs_layout_passes=False,
                                           use_tc_tiling_on_sc=False),
      scratch_types=[
          pltpu.VMEM((_BPW,), jnp.int32),
          pltpu.VMEM((_BPW,), jnp.int32),
          pltpu.VMEM((_BPW, _EMB), jnp.float32),
          pltpu.VMEM((_BPW, _EMB), jnp.float32),
          pltpu.VMEM((_BPW,), jnp.float32),
          pltpu.VMEM_SHARED((_NS * _BPW, _EMB), jnp.float32),
          pltpu.SemaphoreType.DMA,
      ],
  )(d, w, wpt, docs, ctxs_t, y_t)


def kernel(D, W, WP, ctxs, docs, y):
  docs_i = docs.astype(jnp.int32)
  ctxs_t = ctxs.astype(jnp.int32).T.reshape(-1)
  y_t = y.astype(jnp.int32).T.reshape(-1)
  wpt = WP.T
  out_flat = _dmm_call(D, W, wpt, docs_i, ctxs_t, y_t)
  return out_flat.reshape(_S, _B).T
